# Initial kernel scaffold; baseline (speedup 1.0000x reference)
import jax
import jax.numpy as jnp
from jax.experimental import pallas as pl


def kernel(x, edge_index, W, b):
    raise NotImplementedError("write your pallas kernel here")


# speedup vs baseline: 6.1893x; 6.1893x over previous
import functools

import jax
import jax.numpy as jnp
from jax import lax
from jax.experimental import pallas as pl
from jax.experimental.pallas import tpu as pltpu
from jax.experimental.pallas import tpu_sc as plsc

N = 2048
E = 32768
C = 128

NC = 2
NS = 16
NW = NC * NS
EPW = E // NW
SPW = N // NW
CHUNK = 128
RCHUNK = EPW // CHUNK
NCHUNK = RCHUNK + 1
TRASH = N
DEG_ROWS = 2304
AGG_ROWS = 2560

_MESH = plsc.VectorSubcoreMesh(core_axis_name="c", subcore_axis_name="s")


def _fill_tail_chunk(idx_buf, wid, tail_value):
    iota16 = lax.iota(jnp.int32, 16)
    base = wid * SPW
    for k in range(SPW // 16):
        idx_buf[RCHUNK, pl.ds(k * 16, 16)] = base + k * 16 + iota16
    for k in range(SPW // 16, CHUNK // 16):
        idx_buf[RCHUNK, pl.ds(k * 16, 16)] = jnp.full((16,), tail_value, jnp.int32)


@functools.partial(
    pl.kernel,
    out_type=jax.ShapeDtypeStruct((NC, N), jnp.float32),
    mesh=_MESH,
    scratch_types=[
        pltpu.VMEM((NCHUNK, CHUNK), jnp.int32),
        pltpu.VMEM((CHUNK,), jnp.float32),
        pltpu.VMEM((DEG_ROWS // NS,), jnp.float32),
        pltpu.VMEM_SHARED((DEG_ROWS,), jnp.float32),
    ],
)
def _deg_call(row_hbm, deg_out, row_buf, ones_v, tmp_v, deg_sh):
    c = lax.axis_index("c")
    s = lax.axis_index("s")
    wid = c * NS + s
    for k in range(CHUNK // 16):
        ones_v[pl.ds(k * 16, 16)] = jnp.ones((16,), jnp.float32)
    zchunk = DEG_ROWS // NS
    for k in range(zchunk // 16):
        tmp_v[pl.ds(k * 16, 16)] = jnp.zeros((16,), jnp.float32)
    pltpu.sync_copy(tmp_v, deg_sh.at[pl.ds(s * zchunk, zchunk)])
    pltpu.sync_copy(row_hbm.at[pl.ds(wid * RCHUNK, RCHUNK)],
                    row_buf.at[pl.ds(0, RCHUNK)])
    _fill_tail_chunk(row_buf, wid, TRASH)
    plsc.subcore_barrier()
    for g in range(NCHUNK):
        pltpu.sync_copy(ones_v, deg_sh.at[row_buf.at[g]], add=True)
    plsc.subcore_barrier()
    opw = N // NS
    pltpu.sync_copy(deg_sh.at[pl.ds(s * opw, opw)], tmp_v.at[pl.ds(0, opw)])
    pltpu.sync_copy(tmp_v.at[pl.ds(0, opw)], deg_out.at[c, pl.ds(s * opw, opw)])


@functools.partial(
    pl.kernel,
    out_type=jax.ShapeDtypeStruct((NC, N, C), jnp.float32),
    mesh=_MESH,
    scratch_types=[
        pltpu.VMEM((NCHUNK, CHUNK), jnp.int32),
        pltpu.VMEM((NCHUNK, CHUNK), jnp.int32),
        pltpu.VMEM((CHUNK, C), jnp.float32),
        pltpu.VMEM((16, C), jnp.float32),
        pltpu.VMEM_SHARED((AGG_ROWS, C), jnp.float32),
        pltpu.SemaphoreType.DMA,
    ],
)
def _agg_call(hs_hbm, col_hbm, row_hbm, agg_out,
              col_buf, row_buf, gbuf, zrow, agg_sh, sem):
    c = lax.axis_index("c")
    s = lax.axis_index("s")
    wid = c * NS + s
    for r in range(16):
        for k in range(C // 16):
            zrow[r, pl.ds(k * 16, 16)] = jnp.zeros((16,), jnp.float32)
    zrows = AGG_ROWS // NS
    for j in range(zrows // 16):
        pltpu.sync_copy(zrow, agg_sh.at[pl.ds(s * zrows + j * 16, 16)])
    pltpu.sync_copy(col_hbm.at[pl.ds(wid * RCHUNK, RCHUNK)],
                    col_buf.at[pl.ds(0, RCHUNK)])
    pltpu.sync_copy(row_hbm.at[pl.ds(wid * RCHUNK, RCHUNK)],
                    row_buf.at[pl.ds(0, RCHUNK)])
    _fill_tail_chunk(col_buf, wid, 0)
    _fill_tail_chunk(row_buf, wid, TRASH)
    plsc.subcore_barrier()
    for g in range(NCHUNK):
        pltpu.async_copy(hs_hbm.at[col_buf.at[g]], gbuf, sem).wait()
        pltpu.sync_copy(gbuf, agg_sh.at[row_buf.at[g]], add=True)
    plsc.subcore_barrier()
    opw = N // NS
    pltpu.sync_copy(agg_sh.at[pl.ds(s * opw, opw)], gbuf)
    pltpu.sync_copy(gbuf, agg_out.at[c, pl.ds(s * opw, opw)])


def _linear_body(x_ref, w_ref, b_ref, degp_ref, hs_ref, isq_ref):
    deg = degp_ref[0] + degp_ref[1]
    isq = lax.rsqrt(deg)
    h = lax.dot_general(x_ref[...], w_ref[...],
                        (((1,), (1,)), ((), ())),
                        preferred_element_type=jnp.float32)
    hs_ref[...] = (h + b_ref[...]) * isq
    isq_ref[...] = isq


def _combine_body(aggp_ref, isq_ref, out_ref):
    out_ref[...] = (aggp_ref[0] + aggp_ref[1]) * isq_ref[...]


def kernel(x, edge_index, W, b):
    row = edge_index[0].reshape(E // CHUNK, CHUNK)
    col = edge_index[1].reshape(E // CHUNK, CHUNK)

    deg_p = _deg_call(row)

    hs, isq = pl.pallas_call(
        _linear_body,
        out_shape=[
            jax.ShapeDtypeStruct((N, C), jnp.float32),
            jax.ShapeDtypeStruct((N, 1), jnp.float32),
        ],
    )(x, W, b.reshape(1, C), deg_p.reshape(NC, N, 1))

    agg_p = _agg_call(hs, col, row)

    out = pl.pallas_call(
        _combine_body,
        out_shape=jax.ShapeDtypeStruct((N, C), jnp.float32),
    )(agg_p, isq)
    return out
